# Initial kernel scaffold; baseline (speedup 1.0000x reference)
import jax
import jax.numpy as jnp
from jax.experimental import pallas as pl


def kernel(pos, edge_index, W_l0, b_l0, W_r0, W_l1, b_l1, W_r1, W_l2, b_l2, W_r2):
    raise NotImplementedError("write your pallas kernel here")


# speedup vs baseline: 3.6426x; 3.6426x over previous
import jax
import jax.numpy as jnp
from jax import lax
from jax.experimental import pallas as pl
from jax.experimental.pallas import tpu as pltpu
from jax.experimental.pallas import tpu_sc as plsc

N = 50000
N2 = 50176
E = 800000
HID = 128
OUT = 68

NC = 2
NS = 16
K = 80
NB = E // (NS * K)
NBC = 25
NCH = NB // NBC
RPT = N2 // NS
ZROWS = 112


def _make_seg_sum(core_slices, dc, n_in):
    mesh = plsc.VectorSubcoreMesh(core_axis_name="c", subcore_axis_name="s",
                                  num_cores=NC, num_subcores=NS)

    def body(*refs):
        x_refs = list(refs[:n_in])
        edges_ref = refs[n_in]
        out_refs = list(refs[n_in + 1:2 * n_in + 1])
        src_v, dst_v, rowbuf, zbuf, acc, sem = refs[2 * n_in + 1:]

        c = lax.axis_index("c")
        s = lax.axis_index("s")

        zv = jnp.zeros((16,), jnp.float32)

        def _zinit(i, _):
            for j in range(dc // 16):
                zbuf[i, pl.ds(j * 16, 16)] = zv
            return 0

        lax.fori_loop(0, ZROWS, _zinit, 0)

        for core in range(NC):
            slices = core_slices[core]
            if not slices:
                continue

            @pl.when(c == core)
            def _():
                for f in slices:
                    x_hbm = x_refs[f]
                    out_hbm = out_refs[f]
                    for z in range(RPT // ZROWS):
                        pltpu.sync_copy(
                            zbuf, acc.at[pl.ds(s * RPT + z * ZROWS, ZROWS)])
                    plsc.subcore_barrier()

                    def _chunk(ch, _):
                        pltpu.sync_copy(
                            edges_ref.at[0, s, pl.ds(ch * NBC, NBC)], src_v)
                        pltpu.sync_copy(
                            edges_ref.at[1, s, pl.ds(ch * NBC, NBC)], dst_v)

                        def _batch(b, _):
                            pltpu.async_copy(
                                x_hbm.at[src_v.at[b]], rowbuf, sem).wait()
                            pltpu.sync_copy(rowbuf, acc.at[dst_v.at[b]],
                                            add=True)
                            return 0

                        lax.fori_loop(0, NBC, _batch, 0)
                        return 0

                    lax.fori_loop(0, NCH, _chunk, 0)
                    plsc.subcore_barrier()

                    pltpu.sync_copy(acc.at[pl.ds(s * RPT, RPT)],
                                    out_hbm.at[pl.ds(s * RPT, RPT)])
                    plsc.subcore_barrier()

    return pl.kernel(
        body,
        out_type=[jax.ShapeDtypeStruct((N2, dc), jnp.float32)] * n_in,
        mesh=mesh,
        compiler_params=pltpu.CompilerParams(use_tc_tiling_on_sc=False),
        scratch_types=[
            pltpu.VMEM((NBC, K), jnp.int32),
            pltpu.VMEM((NBC, K), jnp.int32),
            pltpu.VMEM((K, dc), jnp.float32),
            pltpu.VMEM((ZROWS, dc), jnp.float32),
            pltpu.VMEM_SHARED((N2, dc), jnp.float32),
            pltpu.SemaphoreType.DMA,
        ],
    )


BN = 1000
GRID = N // BN


def _tc_layer0(agg16, pos, wl, b, wr):
    def body(agg_ref, pos_ref, wl_ref, b_ref, wr_ref, inv_ref, *h_refs):
        agg = agg_ref[...]
        inv = 1.0 / jnp.maximum(agg[:, 3:4], 1.0)
        mean3 = agg[:, 0:4] * inv
        h = (jnp.dot(mean3, wl_ref[...], preferred_element_type=jnp.float32)
             + b_ref[...]
             + jnp.dot(pos_ref[...], wr_ref[...],
                       preferred_element_type=jnp.float32))
        h = jnp.maximum(h, 0.0)
        inv_ref[...] = inv
        for f in range(4):
            h_refs[f][...] = h[:, 32 * f:32 * (f + 1)]

    return pl.pallas_call(
        body,
        grid=(GRID,),
        in_specs=[
            pl.BlockSpec((BN, 16), lambda i: (i, 0)),
            pl.BlockSpec((BN, 4), lambda i: (i, 0)),
            pl.BlockSpec((4, HID), lambda i: (0, 0)),
            pl.BlockSpec((1, HID), lambda i: (0, 0)),
            pl.BlockSpec((4, HID), lambda i: (0, 0)),
        ],
        out_specs=[pl.BlockSpec((BN, 1), lambda i: (i, 0))]
        + [pl.BlockSpec((BN, 32), lambda i: (i, 0)) for _ in range(4)],
        out_shape=[jax.ShapeDtypeStruct((N, 1), jnp.float32)]
        + [jax.ShapeDtypeStruct((N, 32), jnp.float32) for _ in range(4)],
    )(agg16, pos, wl, b, wr)


def _tc_mid(aggs, inv, hs, wl, b, wr, relu, dout):
    nslice_out = dout // 32 if dout % 32 == 0 else None

    def body(*refs):
        agg_refs = refs[0:4]
        inv_ref = refs[4]
        h_refs = refs[5:9]
        wl_ref, b_ref, wr_ref = refs[9:12]
        out_refs = refs[12:]
        inv_v = inv_ref[...]
        mean = jnp.concatenate([r[...] for r in agg_refs], axis=1) * inv_v
        h = jnp.concatenate([r[...] for r in h_refs], axis=1)
        o = (jnp.dot(mean, wl_ref[...], preferred_element_type=jnp.float32)
             + b_ref[...]
             + jnp.dot(h, wr_ref[...], preferred_element_type=jnp.float32))
        if relu:
            o = jnp.maximum(o, 0.0)
        if nslice_out:
            for f in range(nslice_out):
                out_refs[f][...] = o[:, 32 * f:32 * (f + 1)]
        else:
            out_refs[0][...] = o

    if nslice_out:
        out_specs = [pl.BlockSpec((BN, 32), lambda i: (i, 0))
                     for _ in range(nslice_out)]
        out_shape = [jax.ShapeDtypeStruct((N, 32), jnp.float32)
                     for _ in range(nslice_out)]
    else:
        out_specs = [pl.BlockSpec((BN, dout), lambda i: (i, 0))]
        out_shape = [jax.ShapeDtypeStruct((N, dout), jnp.float32)]

    return pl.pallas_call(
        body,
        grid=(GRID,),
        in_specs=[pl.BlockSpec((BN, 32), lambda i: (i, 0)) for _ in range(4)]
        + [pl.BlockSpec((BN, 1), lambda i: (i, 0))]
        + [pl.BlockSpec((BN, 32), lambda i: (i, 0)) for _ in range(4)]
        + [
            pl.BlockSpec((HID, dout), lambda i: (0, 0)),
            pl.BlockSpec((1, dout), lambda i: (0, 0)),
            pl.BlockSpec((HID, dout), lambda i: (0, 0)),
        ],
        out_specs=out_specs,
        out_shape=out_shape,
    )(*aggs, inv, *hs, wl, b, wr)


def kernel(pos, edge_index, W_l0, b_l0, W_r0, W_l1, b_l1, W_r1,
           W_l2, b_l2, W_r2):
    edges = edge_index.reshape(2, NS, NB, K)
    ones = jnp.ones((N, 1), jnp.float32)
    xaug = jnp.concatenate(
        [pos, ones, jnp.zeros((N, 12), jnp.float32)], axis=1)
    pos4 = jnp.concatenate([pos, jnp.zeros((N, 1), jnp.float32)], axis=1)
    wl0 = jnp.concatenate([W_l0, jnp.zeros((1, HID), jnp.float32)], axis=0)
    wr0 = jnp.concatenate([W_r0, jnp.zeros((1, HID), jnp.float32)], axis=0)

    (agg16,) = _make_seg_sum(([0], []), 16, 1)(xaug, edges)
    inv, h10, h11, h12, h13 = _tc_layer0(agg16, pos4, wl0,
                                         b_l0.reshape(1, HID), wr0)
    h1s = [h10, h11, h12, h13]

    seg128 = _make_seg_sum(([0, 2], [1, 3]), 32, 4)
    agg1 = seg128(*h1s, edges)
    h2s = _tc_mid(agg1, inv, h1s, W_l1, b_l1.reshape(1, HID), W_r1,
                  relu=True, dout=HID)
    agg2 = seg128(*h2s, edges)
    (out,) = _tc_mid(agg2, inv, h2s, W_l2, b_l2.reshape(1, OUT), W_r2,
                     relu=False, dout=OUT)
    return out


# speedup vs baseline: 7.4702x; 2.0508x over previous
import jax
import jax.numpy as jnp
from jax import lax
from jax.experimental import pallas as pl
from jax.experimental.pallas import tpu as pltpu
from jax.experimental.pallas import tpu_sc as plsc

N = 50000
N2 = 50176
E = 800000
HID = 128
OUT = 68

NC = 2
NS = 16
K = 80
NB = E // (NS * K)
NBC = 25
NCH = NB // NBC
RPT = N2 // NS
ZROWS = 112


def _make_seg_sum(core_slices, dc, n_in):
    mesh = plsc.VectorSubcoreMesh(core_axis_name="c", subcore_axis_name="s",
                                  num_cores=NC, num_subcores=NS)

    def body(*refs):
        x_refs = list(refs[:n_in])
        edges_ref = refs[n_in]
        out_refs = list(refs[n_in + 1:2 * n_in + 1])
        src_v, dst_v, rowbuf, zbuf, acc, gsem, ssem = refs[2 * n_in + 1:]

        c = lax.axis_index("c")
        s = lax.axis_index("s")

        zv = jnp.zeros((16,), jnp.float32)

        def _zinit(i, _):
            for j in range(dc // 16):
                zbuf[i, pl.ds(j * 16, 16)] = zv
            return 0

        lax.fori_loop(0, ZROWS, _zinit, 0)

        for core in range(NC):
            slices = core_slices[core]
            if not slices:
                continue

            @pl.when(c == core)
            def _():
                for f in slices:
                    x_hbm = x_refs[f]
                    out_hbm = out_refs[f]
                    for z in range(RPT // ZROWS):
                        pltpu.sync_copy(
                            zbuf, acc.at[pl.ds(s * RPT + z * ZROWS, ZROWS)])
                    plsc.subcore_barrier()

                    def _gather(j):
                        return pltpu.async_copy(
                            x_hbm.at[src_v.at[j]], rowbuf.at[j % 4], gsem)

                    def _scatter(j):
                        return pltpu.async_copy(
                            rowbuf.at[j % 4], acc.at[dst_v.at[j]], ssem,
                            add=True)

                    def _chunk(ch, _):
                        pltpu.sync_copy(
                            edges_ref.at[0, s, pl.ds(ch * NBC, NBC)], src_v)
                        pltpu.sync_copy(
                            edges_ref.at[1, s, pl.ds(ch * NBC, NBC)], dst_v)

                        for j in range(min(3, NBC)):
                            _gather(j)
                        for j in range(NBC):
                            if j >= 1:
                                pltpu.make_async_copy(
                                    rowbuf.at[(j - 1) % 4],
                                    acc.at[dst_v.at[j - 1]], ssem).wait()
                            if j + 3 < NBC:
                                _gather(j + 3)
                            pltpu.make_async_copy(
                                x_hbm.at[src_v.at[j]], rowbuf.at[j % 4],
                                gsem).wait()
                            _scatter(j)
                        pltpu.make_async_copy(
                            rowbuf.at[(NBC - 1) % 4],
                            acc.at[dst_v.at[NBC - 1]], ssem).wait()
                        return 0

                    lax.fori_loop(0, NCH, _chunk, 0)
                    plsc.subcore_barrier()

                    pltpu.sync_copy(acc.at[pl.ds(s * RPT, RPT)],
                                    out_hbm.at[pl.ds(s * RPT, RPT)])
                    plsc.subcore_barrier()

    return pl.kernel(
        body,
        out_type=[jax.ShapeDtypeStruct((N2, dc), jnp.float32)] * n_in,
        mesh=mesh,
        compiler_params=pltpu.CompilerParams(use_tc_tiling_on_sc=False),
        scratch_types=[
            pltpu.VMEM((NBC, K), jnp.int32),
            pltpu.VMEM((NBC, K), jnp.int32),
            pltpu.VMEM((4, K, dc), jnp.float32),
            pltpu.VMEM((ZROWS, dc), jnp.float32),
            pltpu.VMEM_SHARED((N2, dc), jnp.float32),
            pltpu.SemaphoreType.DMA,
            pltpu.SemaphoreType.DMA,
        ],
    )


BN = 1000
GRID = N // BN


def _tc_layer0(agg16, pos, wl, b, wr):
    def body(agg_ref, pos_ref, wl_ref, b_ref, wr_ref, inv_ref, *h_refs):
        agg = agg_ref[...]
        inv = 1.0 / jnp.maximum(agg[:, 3:4], 1.0)
        mean3 = agg[:, 0:4] * inv
        h = (jnp.dot(mean3, wl_ref[...], preferred_element_type=jnp.float32)
             + b_ref[...]
             + jnp.dot(pos_ref[...], wr_ref[...],
                       preferred_element_type=jnp.float32))
        h = jnp.maximum(h, 0.0)
        inv_ref[...] = inv
        for f in range(4):
            h_refs[f][...] = h[:, 32 * f:32 * (f + 1)]

    return pl.pallas_call(
        body,
        grid=(GRID,),
        in_specs=[
            pl.BlockSpec((BN, 16), lambda i: (i, 0)),
            pl.BlockSpec((BN, 4), lambda i: (i, 0)),
            pl.BlockSpec((4, HID), lambda i: (0, 0)),
            pl.BlockSpec((1, HID), lambda i: (0, 0)),
            pl.BlockSpec((4, HID), lambda i: (0, 0)),
        ],
        out_specs=[pl.BlockSpec((BN, 1), lambda i: (i, 0))]
        + [pl.BlockSpec((BN, 32), lambda i: (i, 0)) for _ in range(4)],
        out_shape=[jax.ShapeDtypeStruct((N, 1), jnp.float32)]
        + [jax.ShapeDtypeStruct((N, 32), jnp.float32) for _ in range(4)],
    )(agg16, pos, wl, b, wr)


def _tc_mid(aggs, inv, hs, wl, b, wr, relu, dout):
    nslice_out = dout // 32 if dout % 32 == 0 else None

    def body(*refs):
        agg_refs = refs[0:4]
        inv_ref = refs[4]
        h_refs = refs[5:9]
        wl_ref, b_ref, wr_ref = refs[9:12]
        out_refs = refs[12:]
        inv_v = inv_ref[...]
        mean = jnp.concatenate([r[...] for r in agg_refs], axis=1) * inv_v
        h = jnp.concatenate([r[...] for r in h_refs], axis=1)
        o = (jnp.dot(mean, wl_ref[...], preferred_element_type=jnp.float32)
             + b_ref[...]
             + jnp.dot(h, wr_ref[...], preferred_element_type=jnp.float32))
        if relu:
            o = jnp.maximum(o, 0.0)
        if nslice_out:
            for f in range(nslice_out):
                out_refs[f][...] = o[:, 32 * f:32 * (f + 1)]
        else:
            out_refs[0][...] = o

    if nslice_out:
        out_specs = [pl.BlockSpec((BN, 32), lambda i: (i, 0))
                     for _ in range(nslice_out)]
        out_shape = [jax.ShapeDtypeStruct((N, 32), jnp.float32)
                     for _ in range(nslice_out)]
    else:
        out_specs = [pl.BlockSpec((BN, dout), lambda i: (i, 0))]
        out_shape = [jax.ShapeDtypeStruct((N, dout), jnp.float32)]

    return pl.pallas_call(
        body,
        grid=(GRID,),
        in_specs=[pl.BlockSpec((BN, 32), lambda i: (i, 0)) for _ in range(4)]
        + [pl.BlockSpec((BN, 1), lambda i: (i, 0))]
        + [pl.BlockSpec((BN, 32), lambda i: (i, 0)) for _ in range(4)]
        + [
            pl.BlockSpec((HID, dout), lambda i: (0, 0)),
            pl.BlockSpec((1, dout), lambda i: (0, 0)),
            pl.BlockSpec((HID, dout), lambda i: (0, 0)),
        ],
        out_specs=out_specs,
        out_shape=out_shape,
    )(*aggs, inv, *hs, wl, b, wr)


def kernel(pos, edge_index, W_l0, b_l0, W_r0, W_l1, b_l1, W_r1,
           W_l2, b_l2, W_r2):
    edges = edge_index.reshape(2, NS, NB, K)
    ones = jnp.ones((N, 1), jnp.float32)
    xaug = jnp.concatenate(
        [pos, ones, jnp.zeros((N, 12), jnp.float32)], axis=1)
    pos4 = jnp.concatenate([pos, jnp.zeros((N, 1), jnp.float32)], axis=1)
    wl0 = jnp.concatenate([W_l0, jnp.zeros((1, HID), jnp.float32)], axis=0)
    wr0 = jnp.concatenate([W_r0, jnp.zeros((1, HID), jnp.float32)], axis=0)

    (agg16,) = _make_seg_sum(([0], []), 16, 1)(xaug, edges)
    inv, h10, h11, h12, h13 = _tc_layer0(agg16, pos4, wl0,
                                         b_l0.reshape(1, HID), wr0)
    h1s = [h10, h11, h12, h13]

    seg128 = _make_seg_sum(([0, 2], [1, 3]), 32, 4)
    agg1 = seg128(*h1s, edges)
    h2s = _tc_mid(agg1, inv, h1s, W_l1, b_l1.reshape(1, HID), W_r1,
                  relu=True, dout=HID)
    agg2 = seg128(*h2s, edges)
    (out,) = _tc_mid(agg2, inv, h2s, W_l2, b_l2.reshape(1, OUT), W_r2,
                     relu=False, dout=OUT)
    return out


# speedup vs baseline: 8.6018x; 1.1515x over previous
import jax
import jax.numpy as jnp
from jax import lax
from jax.experimental import pallas as pl
from jax.experimental.pallas import tpu as pltpu
from jax.experimental.pallas import tpu_sc as plsc

N = 50000
N2 = 50176
E = 800000
HID = 128
OUT = 68

NC = 2
NS = 16
K = 128
EPT = 50048
NB = EPT // K
NBC = 23
NCH = NB // NBC
RPT = N2 // NS
ZROWS = 112
TRASH = N2 - 1


def _make_seg_sum(jobs, dc, n_in, n_out):
    mesh = plsc.VectorSubcoreMesh(core_axis_name="c", subcore_axis_name="s",
                                  num_cores=NC, num_subcores=NS)

    def body(*refs):
        x_refs = list(refs[:n_in])
        edges_ref = refs[n_in]
        out_refs = list(refs[n_in + 1:n_in + 1 + n_out])
        src_v, dst_v, rowbuf, zbuf, acc, gsem, ssem = refs[n_in + 1 + n_out:]

        c = lax.axis_index("c")
        s = lax.axis_index("s")

        zv = jnp.zeros((16,), jnp.float32)

        def _zinit(i, _):
            for j in range(dc // 16):
                zbuf[i, pl.ds(j * 16, 16)] = zv
            return 0

        lax.fori_loop(0, ZROWS, _zinit, 0)

        for core in range(NC):
            if not jobs[core]:
                continue

            @pl.when(c == core)
            def _():
                for x_idx, out_idx, ch_lo, ch_hi in jobs[core]:
                    x_hbm = x_refs[x_idx]
                    out_hbm = out_refs[out_idx]
                    for z in range(RPT // ZROWS):
                        pltpu.sync_copy(
                            zbuf, acc.at[pl.ds(s * RPT + z * ZROWS, ZROWS)])
                    plsc.subcore_barrier()

                    def _gather(j):
                        pltpu.async_copy(
                            x_hbm.at[src_v.at[j]], rowbuf.at[j % 4], gsem)

                    def _scatter(j):
                        pltpu.async_copy(
                            rowbuf.at[j % 4], acc.at[dst_v.at[j]], ssem,
                            add=True)

                    def _wait_scatter(j):
                        pltpu.make_async_copy(
                            rowbuf.at[j % 4], acc.at[dst_v.at[j]],
                            ssem).wait()

                    def _chunk(ch, _):
                        pltpu.sync_copy(
                            edges_ref.at[0, s, pl.ds(ch * NBC, NBC)], src_v)
                        pltpu.sync_copy(
                            edges_ref.at[1, s, pl.ds(ch * NBC, NBC)], dst_v)

                        for j in range(min(2, NBC)):
                            _gather(j)
                        for j in range(NBC):
                            if j >= 2:
                                _wait_scatter(j - 2)
                            if j + 2 < NBC:
                                _gather(j + 2)
                            pltpu.make_async_copy(
                                x_hbm.at[src_v.at[j]], rowbuf.at[j % 4],
                                gsem).wait()
                            _scatter(j)
                        _wait_scatter(NBC - 2)
                        _wait_scatter(NBC - 1)
                        return 0

                    lax.fori_loop(ch_lo, ch_hi, _chunk, 0)
                    plsc.subcore_barrier()

                    pltpu.sync_copy(acc.at[pl.ds(s * RPT, RPT)],
                                    out_hbm.at[pl.ds(s * RPT, RPT)])
                    plsc.subcore_barrier()

    return pl.kernel(
        body,
        out_type=[jax.ShapeDtypeStruct((N2, dc), jnp.float32)] * n_out,
        mesh=mesh,
        compiler_params=pltpu.CompilerParams(use_tc_tiling_on_sc=False),
        scratch_types=[
            pltpu.VMEM((NBC, K), jnp.int32),
            pltpu.VMEM((NBC, K), jnp.int32),
            pltpu.VMEM((4, K, dc), jnp.float32),
            pltpu.VMEM((ZROWS, dc), jnp.float32),
            pltpu.VMEM_SHARED((N2, dc), jnp.float32),
            pltpu.SemaphoreType.DMA,
            pltpu.SemaphoreType.DMA,
        ],
    )


BN = 1000
GRID = N // BN


def _tc_layer0(agg16a, agg16b, pos, wl, b, wr):
    def body(agga_ref, aggb_ref, pos_ref, wl_ref, b_ref, wr_ref,
             inv_ref, *h_refs):
        agg = agga_ref[...] + aggb_ref[...]
        inv = 1.0 / jnp.maximum(agg[:, 3:4], 1.0)
        mean3 = agg[:, 0:4] * inv
        h = (jnp.dot(mean3, wl_ref[...], preferred_element_type=jnp.float32)
             + b_ref[...]
             + jnp.dot(pos_ref[...], wr_ref[...],
                       preferred_element_type=jnp.float32))
        h = jnp.maximum(h, 0.0)
        inv_ref[...] = inv
        for f in range(4):
            h_refs[f][...] = h[:, 32 * f:32 * (f + 1)]

    return pl.pallas_call(
        body,
        grid=(GRID,),
        in_specs=[
            pl.BlockSpec((BN, 16), lambda i: (i, 0)),
            pl.BlockSpec((BN, 16), lambda i: (i, 0)),
            pl.BlockSpec((BN, 4), lambda i: (i, 0)),
            pl.BlockSpec((4, HID), lambda i: (0, 0)),
            pl.BlockSpec((1, HID), lambda i: (0, 0)),
            pl.BlockSpec((4, HID), lambda i: (0, 0)),
        ],
        out_specs=[pl.BlockSpec((BN, 1), lambda i: (i, 0))]
        + [pl.BlockSpec((BN, 32), lambda i: (i, 0)) for _ in range(4)],
        out_shape=[jax.ShapeDtypeStruct((N, 1), jnp.float32)]
        + [jax.ShapeDtypeStruct((N, 32), jnp.float32) for _ in range(4)],
    )(agg16a, agg16b, pos, wl, b, wr)


def _tc_mid(aggs, inv, hs, wl, b, wr, relu, dout):
    nslice_out = dout // 32 if dout % 32 == 0 else None

    def body(*refs):
        agg_refs = refs[0:4]
        inv_ref = refs[4]
        h_refs = refs[5:9]
        wl_ref, b_ref, wr_ref = refs[9:12]
        out_refs = refs[12:]
        inv_v = inv_ref[...]
        mean = jnp.concatenate([r[...] for r in agg_refs], axis=1) * inv_v
        h = jnp.concatenate([r[...] for r in h_refs], axis=1)
        o = (jnp.dot(mean, wl_ref[...], preferred_element_type=jnp.float32)
             + b_ref[...]
             + jnp.dot(h, wr_ref[...], preferred_element_type=jnp.float32))
        if relu:
            o = jnp.maximum(o, 0.0)
        if nslice_out:
            for f in range(nslice_out):
                out_refs[f][...] = o[:, 32 * f:32 * (f + 1)]
        else:
            out_refs[0][...] = o

    if nslice_out:
        out_specs = [pl.BlockSpec((BN, 32), lambda i: (i, 0))
                     for _ in range(nslice_out)]
        out_shape = [jax.ShapeDtypeStruct((N, 32), jnp.float32)
                     for _ in range(nslice_out)]
    else:
        out_specs = [pl.BlockSpec((BN, dout), lambda i: (i, 0))]
        out_shape = [jax.ShapeDtypeStruct((N, dout), jnp.float32)]

    return pl.pallas_call(
        body,
        grid=(GRID,),
        in_specs=[pl.BlockSpec((BN, 32), lambda i: (i, 0)) for _ in range(4)]
        + [pl.BlockSpec((BN, 1), lambda i: (i, 0))]
        + [pl.BlockSpec((BN, 32), lambda i: (i, 0)) for _ in range(4)]
        + [
            pl.BlockSpec((HID, dout), lambda i: (0, 0)),
            pl.BlockSpec((1, dout), lambda i: (0, 0)),
            pl.BlockSpec((HID, dout), lambda i: (0, 0)),
        ],
        out_specs=out_specs,
        out_shape=out_shape,
    )(*aggs, inv, *hs, wl, b, wr)


def kernel(pos, edge_index, W_l0, b_l0, W_r0, W_l1, b_l1, W_r1,
           W_l2, b_l2, W_r2):
    e3 = edge_index.reshape(2, NS, N)
    pad = jnp.concatenate(
        [jnp.zeros((1, NS, EPT - N), jnp.int32),
         jnp.full((1, NS, EPT - N), TRASH, jnp.int32)], axis=0)
    edges = jnp.concatenate([e3, pad], axis=2).reshape(2, NS, NB, K)
    ones = jnp.ones((N, 1), jnp.float32)
    xaug = jnp.concatenate(
        [pos, ones, jnp.zeros((N, 12), jnp.float32)], axis=1)
    pos4 = jnp.concatenate([pos, jnp.zeros((N, 1), jnp.float32)], axis=1)
    wl0 = jnp.concatenate([W_l0, jnp.zeros((1, HID), jnp.float32)], axis=0)
    wr0 = jnp.concatenate([W_r0, jnp.zeros((1, HID), jnp.float32)], axis=0)

    half = NCH // 2 + 1
    agg16a, agg16b = _make_seg_sum(
        ([(0, 0, 0, half)], [(0, 1, half, NCH)]), 16, 1, 2)(xaug, edges)
    inv, h10, h11, h12, h13 = _tc_layer0(agg16a, agg16b, pos4, wl0,
                                         b_l0.reshape(1, HID), wr0)
    h1s = [h10, h11, h12, h13]

    seg128 = _make_seg_sum(
        ([(0, 0, 0, NCH), (2, 2, 0, NCH)],
         [(1, 1, 0, NCH), (3, 3, 0, NCH)]), 32, 4, 4)
    agg1 = seg128(*h1s, edges)
    h2s = _tc_mid(agg1, inv, h1s, W_l1, b_l1.reshape(1, HID), W_r1,
                  relu=True, dout=HID)
    agg2 = seg128(*h2s, edges)
    (out,) = _tc_mid(agg2, inv, h2s, W_l2, b_l2.reshape(1, OUT), W_r2,
                     relu=False, dout=OUT)
    return out


# speedup vs baseline: 11.1149x; 1.2922x over previous
import jax
import jax.numpy as jnp
from jax import lax
from jax.experimental import pallas as pl
from jax.experimental.pallas import tpu as pltpu
from jax.experimental.pallas import tpu_sc as plsc

N = 50000
N2 = 50176
E = 800000
HID = 128
OUT = 68

NC = 2
NS = 16
K = 128
EPT = 50048
NB = EPT // K
NBC = 23
NCH = NB // NBC
RPT = N2 // NS
ZROWS = 112
TRASH = N2 - 1


def _seg_pipeline(x_src, edges_ref, acc, src_v, dst_v, rowbuf, zbuf,
                  gsem, ssem, s, ch_lo, ch_hi):
    for z in range(RPT // ZROWS):
        pltpu.sync_copy(zbuf, acc.at[pl.ds(s * RPT + z * ZROWS, ZROWS)])
    plsc.subcore_barrier()

    def _gather(j):
        pltpu.async_copy(x_src(src_v.at[j]), rowbuf.at[j % 4], gsem)

    def _scatter(j):
        pltpu.async_copy(rowbuf.at[j % 4], acc.at[dst_v.at[j]], ssem,
                         add=True)

    def _wait_scatter(j):
        pltpu.make_async_copy(rowbuf.at[j % 4], acc.at[dst_v.at[j]],
                              ssem).wait()

    def _chunk(ch, _):
        if isinstance(edges_ref, tuple):
            e_ref, s4_ref, f = edges_ref
            pltpu.sync_copy(s4_ref.at[f, s, pl.ds(ch * NBC, NBC)], src_v)
        else:
            e_ref = edges_ref
            pltpu.sync_copy(e_ref.at[0, s, pl.ds(ch * NBC, NBC)], src_v)
        pltpu.sync_copy(e_ref.at[1, s, pl.ds(ch * NBC, NBC)], dst_v)

        for j in range(min(2, NBC)):
            _gather(j)
        for j in range(NBC):
            if j >= 2:
                _wait_scatter(j - 2)
            if j + 2 < NBC:
                _gather(j + 2)
            pltpu.make_async_copy(x_src(src_v.at[j]), rowbuf.at[j % 4],
                                  gsem).wait()
            _scatter(j)
        _wait_scatter(NBC - 2)
        _wait_scatter(NBC - 1)
        return 0

    lax.fori_loop(ch_lo, ch_hi, _chunk, 0)
    plsc.subcore_barrier()


def _seg_scratch(dc):
    return [
        pltpu.VMEM((NBC, K), jnp.int32),
        pltpu.VMEM((NBC, K), jnp.int32),
        pltpu.VMEM((4, K, dc), jnp.float32),
        pltpu.VMEM((ZROWS, dc), jnp.float32),
        pltpu.VMEM_SHARED((N2, dc), jnp.float32),
        pltpu.SemaphoreType.DMA,
        pltpu.SemaphoreType.DMA,
    ]


def _zinit(zbuf, dc):
    zv = jnp.zeros((16,), jnp.float32)

    def _z(i, _):
        for j in range(dc // 16):
            zbuf[i, pl.ds(j * 16, 16)] = zv
        return 0

    lax.fori_loop(0, ZROWS, _z, 0)


_MESH = plsc.VectorSubcoreMesh(core_axis_name="c", subcore_axis_name="s",
                               num_cores=NC, num_subcores=NS)
_SEG_PARAMS = pltpu.CompilerParams(use_tc_tiling_on_sc=False)


def _make_seg16(half):
    def body(x_ref, edges_ref, outa_ref, outb_ref,
             src_v, dst_v, rowbuf, zbuf, acc, gsem, ssem):
        c = lax.axis_index("c")
        s = lax.axis_index("s")
        _zinit(zbuf, 16)
        for core, out_hbm, lo, hi in ((0, outa_ref, 0, half),
                                      (1, outb_ref, half, NCH)):
            @pl.when(c == core)
            def _():
                _seg_pipeline(lambda i: x_ref.at[i], edges_ref, acc,
                              src_v, dst_v, rowbuf, zbuf, gsem, ssem,
                              s, lo, hi)
                pltpu.sync_copy(acc.at[pl.ds(s * RPT, RPT)],
                                out_hbm.at[pl.ds(s * RPT, RPT)])
                plsc.subcore_barrier()

    return pl.kernel(
        body,
        out_type=[jax.ShapeDtypeStruct((N2, 16), jnp.float32)] * 2,
        mesh=_MESH,
        compiler_params=_SEG_PARAMS,
        scratch_types=_seg_scratch(16),
    )


def _make_seg128():
    def body(x_ref, edges_ref, src4_ref, out_ref,
             src_v, dst_v, rowbuf, zbuf, acc, gsem, ssem):
        c = lax.axis_index("c")
        s = lax.axis_index("s")
        _zinit(zbuf, 32)
        for core in range(NC):
            @pl.when(c == core)
            def _():
                for f in range(core, 4, NC):
                    _seg_pipeline(lambda i: x_ref.at[i],
                                  (edges_ref, src4_ref, f), acc,
                                  src_v, dst_v, rowbuf,
                                  zbuf, gsem, ssem, s, 0, NCH)
                    rows = pl.ds(s * RPT, RPT)
                    pltpu.sync_copy(acc.at[rows],
                                    out_ref.at[rows, pl.ds(32 * f, 32)])
                    plsc.subcore_barrier()

    return pl.kernel(
        body,
        out_type=jax.ShapeDtypeStruct((N2, HID), jnp.float32),
        mesh=_MESH,
        compiler_params=_SEG_PARAMS,
        scratch_types=_seg_scratch(32),
    )


BN = 1000
GRID = N // BN


def _tc_layer0(agg16a, agg16b, pos, wl, b, wr):
    def body(agga_ref, aggb_ref, pos_ref, wl_ref, b_ref, wr_ref,
             inv_ref, h_ref):
        agg = agga_ref[...] + aggb_ref[...]
        inv = 1.0 / jnp.maximum(agg[:, 3:4], 1.0)
        mean3 = agg[:, 0:4] * inv
        h = (jnp.dot(mean3, wl_ref[...], preferred_element_type=jnp.float32)
             + b_ref[...]
             + jnp.dot(pos_ref[...], wr_ref[...],
                       preferred_element_type=jnp.float32))
        inv_ref[...] = inv
        h_ref[...] = jnp.maximum(h, 0.0)

    return pl.pallas_call(
        body,
        grid=(GRID,),
        in_specs=[
            pl.BlockSpec((BN, 16), lambda i: (i, 0)),
            pl.BlockSpec((BN, 16), lambda i: (i, 0)),
            pl.BlockSpec((BN, 4), lambda i: (i, 0)),
            pl.BlockSpec((4, HID), lambda i: (0, 0)),
            pl.BlockSpec((1, HID), lambda i: (0, 0)),
            pl.BlockSpec((4, HID), lambda i: (0, 0)),
        ],
        out_specs=[pl.BlockSpec((BN, 1), lambda i: (i, 0)),
                   pl.BlockSpec((BN, HID), lambda i: (i, 0))],
        out_shape=[jax.ShapeDtypeStruct((N, 1), jnp.float32),
                   jax.ShapeDtypeStruct((N, HID), jnp.float32)],
    )(agg16a, agg16b, pos, wl, b, wr)


def _tc_mid(agg, inv, h, wl, b, wr, relu, dout):
    def body(agg_ref, inv_ref, h_ref, wl_ref, b_ref, wr_ref, out_ref):
        o = (jnp.dot(agg_ref[...], wl_ref[...],
                     preferred_element_type=jnp.float32) * inv_ref[...]
             + b_ref[...]
             + jnp.dot(h_ref[...], wr_ref[...],
                       preferred_element_type=jnp.float32))
        if relu:
            o = jnp.maximum(o, 0.0)
        out_ref[...] = o

    return pl.pallas_call(
        body,
        grid=(GRID,),
        in_specs=[
            pl.BlockSpec((BN, HID), lambda i: (i, 0)),
            pl.BlockSpec((BN, 1), lambda i: (i, 0)),
            pl.BlockSpec((BN, HID), lambda i: (i, 0)),
            pl.BlockSpec((HID, dout), lambda i: (0, 0)),
            pl.BlockSpec((1, dout), lambda i: (0, 0)),
            pl.BlockSpec((HID, dout), lambda i: (0, 0)),
        ],
        out_specs=pl.BlockSpec((BN, dout), lambda i: (i, 0)),
        out_shape=jax.ShapeDtypeStruct((N, dout), jnp.float32),
    )(agg, inv, h, wl, b, wr)


def kernel(pos, edge_index, W_l0, b_l0, W_r0, W_l1, b_l1, W_r1,
           W_l2, b_l2, W_r2):
    e3 = edge_index.reshape(2, NS, N)
    pad = jnp.concatenate(
        [jnp.zeros((1, NS, EPT - N), jnp.int32),
         jnp.full((1, NS, EPT - N), TRASH, jnp.int32)], axis=0)
    edges = jnp.concatenate([e3, pad], axis=2).reshape(2, NS, NB, K)
    ones = jnp.ones((N, 1), jnp.float32)
    xaug = jnp.concatenate(
        [pos, ones, jnp.zeros((N, 12), jnp.float32)], axis=1)
    pos4 = jnp.concatenate([pos, jnp.zeros((N, 1), jnp.float32)], axis=1)
    wl0 = jnp.concatenate([W_l0, jnp.zeros((1, HID), jnp.float32)], axis=0)
    wr0 = jnp.concatenate([W_r0, jnp.zeros((1, HID), jnp.float32)], axis=0)

    src4 = (edges[0] * 4)[None, ...] + jnp.arange(4, dtype=jnp.int32)[
        :, None, None, None]

    agg16a, agg16b = _make_seg16(NCH // 2 + 1)(xaug, edges)
    inv, h1 = _tc_layer0(agg16a, agg16b, pos4, wl0,
                         b_l0.reshape(1, HID), wr0)

    seg128 = _make_seg128()
    agg1 = seg128(h1.reshape(4 * N, 32), edges, src4)
    h2 = _tc_mid(agg1, inv, h1, W_l1, b_l1.reshape(1, HID), W_r1,
                 relu=True, dout=HID)
    agg2 = seg128(h2.reshape(4 * N, 32), edges, src4)
    out = _tc_mid(agg2, inv, h2, W_l2, b_l2.reshape(1, OUT), W_r2,
                  relu=False, dout=OUT)
    return out
